# initial kernel scaffold (unmeasured)
import jax
import jax.numpy as jnp
from jax import lax
from jax.experimental import pallas as pl
from jax.experimental.pallas import tpu as pltpu


def kernel(
    x,
):
    def body(*refs):
        pass

    out_shape = jax.ShapeDtypeStruct(..., jnp.float32)
    return pl.pallas_call(body, out_shape=out_shape)(...)


# baseline (device time: 2129757 ns/iter reference)
import jax
import jax.numpy as jnp
from jax import lax
from jax.experimental import pallas as pl
from jax.experimental.pallas import tpu as pltpu


def kernel(x):
    m, n = x.shape

    def body(x_ref, out_ref, local_sem, send_sem, recv_sem):
        my_x = lax.axis_index("x")
        my_y = lax.axis_index("y")
        my_z = lax.axis_index("z")
        nbr = (1 - my_x, my_y, my_z)

        barrier_sem = pltpu.get_barrier_semaphore()
        pl.semaphore_signal(
            barrier_sem, inc=1, device_id=nbr,
            device_id_type=pl.DeviceIdType.MESH,
        )
        pl.semaphore_wait(barrier_sem, 1)

        rdma = pltpu.make_async_remote_copy(
            src_ref=x_ref,
            dst_ref=out_ref.at[pl.ds(my_x * m, m)],
            send_sem=send_sem,
            recv_sem=recv_sem,
            device_id=nbr,
            device_id_type=pl.DeviceIdType.MESH,
        )
        rdma.start()

        local = pltpu.make_async_copy(
            x_ref, out_ref.at[pl.ds(my_x * m, m)], local_sem
        )
        local.start()
        local.wait()

        rdma.wait()

    return pl.pallas_call(
        body,
        out_shape=jax.ShapeDtypeStruct((2 * m, n), x.dtype),
        in_specs=[pl.BlockSpec(memory_space=pl.ANY)],
        out_specs=pl.BlockSpec(memory_space=pl.ANY),
        scratch_shapes=[
            pltpu.SemaphoreType.DMA,
            pltpu.SemaphoreType.DMA,
            pltpu.SemaphoreType.DMA,
        ],
        compiler_params=pltpu.CompilerParams(collective_id=0),
    )(x)


# device time: 2128296 ns/iter; 1.0007x vs baseline; 1.0007x over previous
import jax
import jax.numpy as jnp
from jax import lax
from jax.experimental import pallas as pl
from jax.experimental.pallas import tpu as pltpu


def kernel(x):
    m, n = x.shape

    K = 16
    rows = m // K

    def body(x_ref, out_ref, local_sem, send_sems, recv_sems):
        my_x = lax.axis_index("x")
        my_y = lax.axis_index("y")
        my_z = lax.axis_index("z")
        nbr = (1 - my_x, my_y, my_z)

        barrier_sem = pltpu.get_barrier_semaphore()
        pl.semaphore_signal(
            barrier_sem, inc=1, device_id=nbr,
            device_id_type=pl.DeviceIdType.MESH,
        )
        pl.semaphore_wait(barrier_sem, 1)

        rdmas = []
        for i in range(K):
            rdma = pltpu.make_async_remote_copy(
                src_ref=x_ref.at[pl.ds(i * rows, rows)],
                dst_ref=out_ref.at[pl.ds(my_x * m + i * rows, rows)],
                send_sem=send_sems.at[i],
                recv_sem=recv_sems.at[i],
                device_id=nbr,
                device_id_type=pl.DeviceIdType.MESH,
            )
            rdma.start()
            rdmas.append(rdma)

        local = pltpu.make_async_copy(
            x_ref, out_ref.at[pl.ds(my_x * m, m)], local_sem
        )
        local.start()
        local.wait()

        for rdma in rdmas:
            rdma.wait()

    return pl.pallas_call(
        body,
        out_shape=jax.ShapeDtypeStruct((2 * m, n), x.dtype),
        in_specs=[pl.BlockSpec(memory_space=pl.ANY)],
        out_specs=pl.BlockSpec(memory_space=pl.ANY),
        scratch_shapes=[
            pltpu.SemaphoreType.DMA,
            pltpu.SemaphoreType.DMA((K,)),
            pltpu.SemaphoreType.DMA((K,)),
        ],
        compiler_params=pltpu.CompilerParams(collective_id=0),
    )(x)


# device time: 478099 ns/iter; 4.4546x vs baseline; 4.4516x over previous
import jax
import jax.numpy as jnp
from jax import lax
from jax.experimental import pallas as pl
from jax.experimental.pallas import tpu as pltpu

C = 16


def kernel(x):
    m, n = x.shape
    h = m // 2
    rc = h // C
    lc = m // C

    def body(x_ref, out_ref, vbuf,
             xsend_sems, xrecv_sems, ysend_sems, yrecv_sems,
             load_sems, store_sems):
        my_x = lax.axis_index("x")
        my_y = lax.axis_index("y")
        my_z = lax.axis_index("z")
        nbr_x = (1 - my_x, my_y, my_z)
        nbr_y = (my_x, 1 - my_y, my_z)

        rm = (1 - my_x) * m
        xrecv_base = rm + my_y * h
        yrecv_base = rm + (1 - my_y) * h

        barrier_sem = pltpu.get_barrier_semaphore()
        for nbr in (nbr_x, nbr_y):
            pl.semaphore_signal(barrier_sem, inc=1, device_id=nbr,
                                device_id_type=pl.DeviceIdType.MESH)
        pl.semaphore_wait(barrier_sem, 2)

        xsends = []
        for c in range(C):
            r = pltpu.make_async_remote_copy(
                src_ref=x_ref.at[pl.ds(my_y * h + c * rc, rc)],
                dst_ref=out_ref.at[pl.ds(my_x * m + my_y * h + c * rc, rc)],
                send_sem=xsend_sems.at[c],
                recv_sem=xrecv_sems.at[c],
                device_id=nbr_x,
                device_id_type=pl.DeviceIdType.MESH,
            )
            r.start()
            xsends.append(r)

        def xrecv_desc(c):
            return pltpu.make_async_remote_copy(
                src_ref=x_ref.at[pl.ds(c * rc, rc)],
                dst_ref=out_ref.at[pl.ds(xrecv_base + c * rc, rc)],
                send_sem=xsend_sems.at[c],
                recv_sem=xrecv_sems.at[c],
                device_id=nbr_x,
                device_id_type=pl.DeviceIdType.MESH,
            )

        yfwds = []
        pending = [None, None]
        for c in range(C):
            xrecv_desc(c).wait_recv()
            fwd = pltpu.make_async_remote_copy(
                src_ref=out_ref.at[pl.ds(xrecv_base + c * rc, rc)],
                dst_ref=out_ref.at[pl.ds(xrecv_base + c * rc, rc)],
                send_sem=ysend_sems.at[c],
                recv_sem=yrecv_sems.at[c],
                device_id=nbr_y,
                device_id_type=pl.DeviceIdType.MESH,
            )
            fwd.start()
            yfwds.append(fwd)

            s = c % 2
            if pending[s] is not None:
                pending[s].wait()
            ld = pltpu.make_async_copy(
                x_ref.at[pl.ds(c * lc, lc)], vbuf.at[s], load_sems.at[s])
            ld.start()
            ld.wait()
            st = pltpu.make_async_copy(
                vbuf.at[s], out_ref.at[pl.ds(my_x * m + c * lc, lc)],
                store_sems.at[s])
            st.start()
            pending[s] = st

        for c in range(C):
            pltpu.make_async_remote_copy(
                src_ref=x_ref.at[pl.ds(c * rc, rc)],
                dst_ref=out_ref.at[pl.ds(yrecv_base + c * rc, rc)],
                send_sem=ysend_sems.at[c],
                recv_sem=yrecv_sems.at[c],
                device_id=nbr_y,
                device_id_type=pl.DeviceIdType.MESH,
            ).wait_recv()
        for r in xsends:
            r.wait_send()
        for r in yfwds:
            r.wait_send()
        for p in pending:
            if p is not None:
                p.wait()

    return pl.pallas_call(
        body,
        out_shape=jax.ShapeDtypeStruct((2 * m, n), x.dtype),
        in_specs=[pl.BlockSpec(memory_space=pl.ANY)],
        out_specs=pl.BlockSpec(memory_space=pl.ANY),
        scratch_shapes=[
            pltpu.VMEM((2, lc, n), x.dtype),
            pltpu.SemaphoreType.DMA((C,)),
            pltpu.SemaphoreType.DMA((C,)),
            pltpu.SemaphoreType.DMA((C,)),
            pltpu.SemaphoreType.DMA((C,)),
            pltpu.SemaphoreType.DMA((2,)),
            pltpu.SemaphoreType.DMA((2,)),
        ],
        compiler_params=pltpu.CompilerParams(collective_id=0),
    )(x)


# device time: 389659 ns/iter; 5.4657x vs baseline; 1.2270x over previous
import jax
import jax.numpy as jnp
from jax import lax
from jax.experimental import pallas as pl
from jax.experimental.pallas import tpu as pltpu

C = 8
C2 = C // 2


def kernel(x):
    m, n = x.shape
    qh = m // 4
    rc = qh // C
    lc = m // C

    def body(x_ref, out_ref, vbuf,
             xs_sems, xr_sems, yds_sems, ydr_sems, zds_sems, zdr_sems,
             y2s_sems, y2r_sems, z2s_sems, z2r_sems,
             load_sems, store_sems):
        my_x = lax.axis_index("x")
        my_y = lax.axis_index("y")
        my_z = lax.axis_index("z")
        nbr_x = (1 - my_x, my_y, my_z)
        nbr_y = (my_x, 1 - my_y, my_z)
        nbr_z = (my_x, my_y, 1 - my_z)

        rm = (1 - my_x) * m
        d_base = rm + (2 * my_y + my_z) * qh
        a_base = rm + (2 * (1 - my_y) + my_z) * qh
        b_base = rm + (2 * my_y + (1 - my_z)) * qh
        g_base = rm + (2 * (1 - my_y) + (1 - my_z)) * qh

        barrier_sem = pltpu.get_barrier_semaphore()
        for nbr in (nbr_x, nbr_y, nbr_z):
            pl.semaphore_signal(barrier_sem, inc=1, device_id=nbr,
                                device_id_type=pl.DeviceIdType.MESH)
        pl.semaphore_wait(barrier_sem, 3)

        xsends = []
        for c in range(C):
            r = pltpu.make_async_remote_copy(
                src_ref=x_ref.at[pl.ds(d_base - rm + c * rc, rc)],
                dst_ref=out_ref.at[pl.ds(my_x * m + (d_base - rm) + c * rc,
                                         rc)],
                send_sem=xs_sems.at[c],
                recv_sem=xr_sems.at[c],
                device_id=nbr_x,
                device_id_type=pl.DeviceIdType.MESH,
            )
            r.start()
            xsends.append(r)

        def recv_only(base, c, recv_sems):
            return pltpu.make_async_remote_copy(
                src_ref=x_ref.at[pl.ds(c * rc, rc)],
                dst_ref=out_ref.at[pl.ds(base + c * rc, rc)],
                send_sem=xs_sems.at[c],
                recv_sem=recv_sems.at[c],
                device_id=nbr_x,
                device_id_type=pl.DeviceIdType.MESH,
            )

        def fwd(src_base, c, send_sems, recv_sems, nbr):
            return pltpu.make_async_remote_copy(
                src_ref=out_ref.at[pl.ds(src_base + c * rc, rc)],
                dst_ref=out_ref.at[pl.ds(src_base + c * rc, rc)],
                send_sem=send_sems.at[c],
                recv_sem=recv_sems.at[c],
                device_id=nbr,
                device_id_type=pl.DeviceIdType.MESH,
            )

        ydf, zdf = [], []
        pending = [None, None]
        for c in range(C):
            recv_only(d_base, c, xr_sems).wait_recv()
            r = fwd(d_base, c, yds_sems, ydr_sems, nbr_y)
            r.start()
            ydf.append(r)
            r = fwd(d_base, c, zds_sems, zdr_sems, nbr_z)
            r.start()
            zdf.append(r)

            s = c % 2
            if pending[s] is not None:
                pending[s].wait()
            ld = pltpu.make_async_copy(
                x_ref.at[pl.ds(c * lc, lc)], vbuf.at[s], load_sems.at[s])
            ld.start()
            ld.wait()
            st = pltpu.make_async_copy(
                vbuf.at[s], out_ref.at[pl.ds(my_x * m + c * lc, lc)],
                store_sems.at[s])
            st.start()
            pending[s] = st

        y2f = []
        for c in range(C2):
            recv_only(b_base, c, zdr_sems).wait_recv()
            r = fwd(b_base, c, y2s_sems, y2r_sems, nbr_y)
            r.start()
            y2f.append(r)
        z2f = []
        for c in range(C2):
            recv_only(a_base, C2 + c, ydr_sems).wait_recv()
            r = pltpu.make_async_remote_copy(
                src_ref=out_ref.at[pl.ds(a_base + (C2 + c) * rc, rc)],
                dst_ref=out_ref.at[pl.ds(a_base + (C2 + c) * rc, rc)],
                send_sem=z2s_sems.at[c],
                recv_sem=z2r_sems.at[c],
                device_id=nbr_z,
                device_id_type=pl.DeviceIdType.MESH,
            )
            r.start()
            z2f.append(r)

        for c in range(C2):
            recv_only(b_base, C2 + c, zdr_sems).wait_recv()
        for c in range(C2):
            recv_only(a_base, c, ydr_sems).wait_recv()
        for c in range(C2):
            recv_only(g_base, c, y2r_sems).wait_recv()
        for c in range(C2):
            pltpu.make_async_remote_copy(
                src_ref=x_ref.at[pl.ds(c * rc, rc)],
                dst_ref=out_ref.at[pl.ds(g_base + (C2 + c) * rc, rc)],
                send_sem=xs_sems.at[c],
                recv_sem=z2r_sems.at[c],
                device_id=nbr_x,
                device_id_type=pl.DeviceIdType.MESH,
            ).wait_recv()
        for r in xsends:
            r.wait_send()
        for r in ydf:
            r.wait_send()
        for r in zdf:
            r.wait_send()
        for r in y2f:
            r.wait_send()
        for r in z2f:
            r.wait_send()
        for p in pending:
            if p is not None:
                p.wait()

    return pl.pallas_call(
        body,
        out_shape=jax.ShapeDtypeStruct((2 * m, n), x.dtype),
        in_specs=[pl.BlockSpec(memory_space=pl.ANY)],
        out_specs=pl.BlockSpec(memory_space=pl.ANY),
        scratch_shapes=[
            pltpu.VMEM((2, m // C, n), x.dtype),
            pltpu.SemaphoreType.DMA((C,)),
            pltpu.SemaphoreType.DMA((C,)),
            pltpu.SemaphoreType.DMA((C,)),
            pltpu.SemaphoreType.DMA((C,)),
            pltpu.SemaphoreType.DMA((C,)),
            pltpu.SemaphoreType.DMA((C,)),
            pltpu.SemaphoreType.DMA((C2,)),
            pltpu.SemaphoreType.DMA((C2,)),
            pltpu.SemaphoreType.DMA((C2,)),
            pltpu.SemaphoreType.DMA((C2,)),
            pltpu.SemaphoreType.DMA((2,)),
            pltpu.SemaphoreType.DMA((2,)),
        ],
        compiler_params=pltpu.CompilerParams(collective_id=0),
    )(x)


# device time: 356050 ns/iter; 5.9816x vs baseline; 1.0944x over previous
import jax
import jax.numpy as jnp
from jax import lax
from jax.experimental import pallas as pl
from jax.experimental.pallas import tpu as pltpu

C = 8

GX_RANGES = [(0, 512), (512, 512), (1024, 512)]
Y2_RANGES = [(1536, 512), (2048, 512), (2560, 256)]
Z2_RANGES = [(2816, 256), (3072, 512), (3584, 512)]


def kernel(x):
    m, n = x.shape
    qh = m // 4
    rc = qh // C
    lc = m // C

    def body(x_ref, out_ref, vbuf,
             xs_sems, xr_sems, gxs_sems, gxr_sems,
             yds_sems, ydr_sems, zds_sems, zdr_sems,
             y2s_sems, y2r_sems, z2s_sems, z2r_sems,
             load_sems, store_sems):
        my_x = lax.axis_index("x")
        my_y = lax.axis_index("y")
        my_z = lax.axis_index("z")
        nbr_x = (1 - my_x, my_y, my_z)
        nbr_y = (my_x, 1 - my_y, my_z)
        nbr_z = (my_x, my_y, 1 - my_z)

        rm = (1 - my_x) * m
        d_base = rm + (2 * my_y + my_z) * qh
        a_base = rm + (2 * (1 - my_y) + my_z) * qh
        b_base = rm + (2 * my_y + (1 - my_z)) * qh
        g_base = rm + (2 * (1 - my_y) + (1 - my_z)) * qh

        barrier_sem = pltpu.get_barrier_semaphore()
        for nbr in (nbr_x, nbr_y, nbr_z):
            pl.semaphore_signal(barrier_sem, inc=1, device_id=nbr,
                                device_id_type=pl.DeviceIdType.MESH)
        pl.semaphore_wait(barrier_sem, 3)

        def send_x(shard_off, rows, send_sem, recv_sem):
            r = pltpu.make_async_remote_copy(
                src_ref=x_ref.at[pl.ds(shard_off, rows)],
                dst_ref=out_ref.at[pl.ds(my_x * m + shard_off, rows)],
                send_sem=send_sem,
                recv_sem=recv_sem,
                device_id=nbr_x,
                device_id_type=pl.DeviceIdType.MESH,
            )
            r.start()
            return r

        def fwd(out_off, rows, send_sem, recv_sem, nbr):
            r = pltpu.make_async_remote_copy(
                src_ref=out_ref.at[pl.ds(out_off, rows)],
                dst_ref=out_ref.at[pl.ds(out_off, rows)],
                send_sem=send_sem,
                recv_sem=recv_sem,
                device_id=nbr,
                device_id_type=pl.DeviceIdType.MESH,
            )
            r.start()
            return r

        def wait_recv(out_off, rows, recv_sem):
            pltpu.make_async_remote_copy(
                src_ref=x_ref.at[pl.ds(0, rows)],
                dst_ref=out_ref.at[pl.ds(out_off, rows)],
                send_sem=recv_sem,
                recv_sem=recv_sem,
                device_id=nbr_x,
                device_id_type=pl.DeviceIdType.MESH,
            ).wait_recv()

        sends = []
        for c in range(C):
            sends.append(send_x(d_base - rm + c * rc,
                                rc, xs_sems.at[c], xr_sems.at[c]))
        for i, (off, rows) in enumerate(GX_RANGES):
            sends.append(send_x(g_base - rm + off,
                                rows, gxs_sems.at[i], gxr_sems.at[i]))

        pending = [None, None]
        for c in range(C):
            wait_recv(d_base + c * rc, rc, xr_sems.at[c])
            sends.append(fwd(d_base + c * rc, rc,
                             yds_sems.at[c], ydr_sems.at[c], nbr_y))
            sends.append(fwd(d_base + c * rc, rc,
                             zds_sems.at[c], zdr_sems.at[c], nbr_z))

            s = c % 2
            if pending[s] is not None:
                pending[s].wait()
            ld = pltpu.make_async_copy(
                x_ref.at[pl.ds(c * lc, lc)], vbuf.at[s], load_sems.at[s])
            ld.start()
            ld.wait()
            st = pltpu.make_async_copy(
                vbuf.at[s], out_ref.at[pl.ds(my_x * m + c * lc, lc)],
                store_sems.at[s])
            st.start()
            pending[s] = st

        zdr_waited = 0
        for i, (off, rows) in enumerate(Y2_RANGES):
            need = (off + rows - 1) // rc + 1
            while zdr_waited < need:
                wait_recv(b_base + zdr_waited * rc, rc,
                          zdr_sems.at[zdr_waited])
                zdr_waited += 1
            sends.append(fwd(b_base + off, rows,
                             y2s_sems.at[i], y2r_sems.at[i], nbr_y))
        ydr_waited = 0
        for i, (off, rows) in enumerate(Z2_RANGES):
            need = (off + rows - 1) // rc + 1
            while ydr_waited < need:
                wait_recv(a_base + ydr_waited * rc, rc,
                          ydr_sems.at[ydr_waited])
                ydr_waited += 1
            sends.append(fwd(a_base + off, rows,
                             z2s_sems.at[i], z2r_sems.at[i], nbr_z))

        for c in range(zdr_waited, C):
            wait_recv(b_base + c * rc, rc, zdr_sems.at[c])
        for c in range(ydr_waited, C):
            wait_recv(a_base + c * rc, rc, ydr_sems.at[c])
        for i, (off, rows) in enumerate(GX_RANGES):
            wait_recv(g_base + off, rows, gxr_sems.at[i])
        for i, (off, rows) in enumerate(Y2_RANGES):
            wait_recv(g_base + off, rows, y2r_sems.at[i])
        for i, (off, rows) in enumerate(Z2_RANGES):
            wait_recv(g_base + off, rows, z2r_sems.at[i])
        for r in sends:
            r.wait_send()
        for p in pending:
            if p is not None:
                p.wait()

    n3 = len(GX_RANGES)
    return pl.pallas_call(
        body,
        out_shape=jax.ShapeDtypeStruct((2 * m, n), x.dtype),
        in_specs=[pl.BlockSpec(memory_space=pl.ANY)],
        out_specs=pl.BlockSpec(memory_space=pl.ANY),
        scratch_shapes=[
            pltpu.VMEM((2, m // C, n), x.dtype),
            pltpu.SemaphoreType.DMA((C,)),
            pltpu.SemaphoreType.DMA((C,)),
            pltpu.SemaphoreType.DMA((n3,)),
            pltpu.SemaphoreType.DMA((n3,)),
            pltpu.SemaphoreType.DMA((C,)),
            pltpu.SemaphoreType.DMA((C,)),
            pltpu.SemaphoreType.DMA((C,)),
            pltpu.SemaphoreType.DMA((C,)),
            pltpu.SemaphoreType.DMA((n3,)),
            pltpu.SemaphoreType.DMA((n3,)),
            pltpu.SemaphoreType.DMA((n3,)),
            pltpu.SemaphoreType.DMA((n3,)),
            pltpu.SemaphoreType.DMA((2,)),
            pltpu.SemaphoreType.DMA((2,)),
        ],
        compiler_params=pltpu.CompilerParams(collective_id=0),
    )(x)
